# baseline (speedup 1.0000x reference)
import functools

import jax
import jax.numpy as jnp
from jax import lax
from jax.experimental import pallas as pl
from jax.experimental.pallas import tpu as pltpu
from jax.experimental.pallas import tpu_sc as plsc

_NC = 2
_NS = 16
_NW = _NC * _NS
_L = 16

_BATCH = 4
_SEQ = 2048
_D = 1024
_S_PER_W = _SEQ // _NW
_CHUNK = 32


def _body(tok_hbm, emb_hbm, pos_hbm, out_hbm, pbuf, ebuf, idxv, sem):
    wid = lax.axis_index("s") * _NC + lax.axis_index("c")
    s0 = wid * _S_PER_W
    pltpu.sync_copy(pos_hbm.at[pl.ds(s0, _S_PER_W)], pbuf)
    for b in range(_BATCH):
        base = b * _SEQ + s0
        pltpu.sync_copy(tok_hbm.at[pl.ds(base, _S_PER_W)], idxv)
        for c in range(_S_PER_W // _CHUNK):
            pltpu.async_copy(
                emb_hbm.at[idxv.at[pl.ds(c * _CHUNK, _CHUNK)]], ebuf, sem
            ).wait()

            def row_add(r, carry, _c=c):
                for j in range(_D // _L):
                    sl = pl.ds(j * _L, _L)
                    ebuf[r, sl] = ebuf[r, sl] + pbuf[_c * _CHUNK + r, sl]
                return carry

            lax.fori_loop(0, _CHUNK, row_add, 0)
            pltpu.sync_copy(ebuf, out_hbm.at[pl.ds(base + c * _CHUNK, _CHUNK)])


_mesh = plsc.VectorSubcoreMesh(core_axis_name="c", subcore_axis_name="s")

_embed = pl.kernel(
    _body,
    out_type=jax.ShapeDtypeStruct((_BATCH * _SEQ, _D), jnp.float32),
    mesh=_mesh,
    scratch_types=[
        pltpu.VMEM((_S_PER_W, _D), jnp.float32),
        pltpu.VMEM((_CHUNK, _D), jnp.float32),
        pltpu.VMEM((_S_PER_W,), jnp.int32),
        pltpu.SemaphoreType.DMA,
    ],
)


@jax.jit
def kernel(token_ids, embed_table, pos_table):
    tok = token_ids.reshape(-1).astype(jnp.int32)
    out = _embed(tok, embed_table, pos_table)
    return out.reshape(_BATCH, _SEQ, _D)
